# baseline (speedup 1.0000x reference)
import functools

import numpy as np
import jax
import jax.numpy as jnp
from jax import lax
from jax.experimental import pallas as pl
from jax.experimental.pallas import tpu as pltpu
from jax.experimental.pallas import tpu_sc as plsc

N = 100
F = 41
HID = 128
NCLS = 2
P = N * (N - 1) // 2

NW = 32
PPW = 160
NPAD = NW * PPW
BLK = 16
NCHK = HID // 16
NBLK = PPW // BLK
WDL = HID + BLK

_i_np, _j_np = np.triu_indices(N, k=1)
_IDXS = np.stack([_i_np, _j_np], axis=1).astype(np.int32)
_I_PAD = np.zeros((NPAD,), np.int32)
_J_PAD = np.zeros((NPAD,), np.int32)
_I_PAD[:P] = _i_np
_J_PAD[:P] = _j_np


def _tc_body(adj, feat, wc0, wc1, wc2, wc3, w1, b1, w2t, b2,
             a_out, b_out, wdl_out):
    adj_v = adj[...]
    deg = jnp.sum(adj_v, axis=1, keepdims=True)
    na = adj_v / jnp.maximum(deg, 1.0)
    h = feat[...]
    for wc in (wc0, wc1, wc2, wc3):
        h = jnp.maximum(
            jnp.dot(jnp.dot(na, h, preferred_element_type=jnp.float32),
                    wc[...], preferred_element_type=jnp.float32), 0.0)
    w1v = w1[...]
    a_out[...] = jnp.dot(h, w1v[:F], preferred_element_type=jnp.float32) + b1[...]
    b_out[...] = jnp.dot(h, w1v[F:], preferred_element_type=jnp.float32)
    wd_row = w2t[0:1, :] - w2t[1:2, :]
    dltb = b2[0, 0] - b2[0, 1]
    wdl_out[...] = jnp.concatenate(
        [wd_row, jnp.full((1, BLK), dltb, jnp.float32)], axis=1)


def _tc_tables(adj, feat, wc0, wc1, wc2, wc3, w1, b1r, w2t, b2r):
    return pl.pallas_call(
        _tc_body,
        out_shape=[
            jax.ShapeDtypeStruct((N, HID), jnp.float32),
            jax.ShapeDtypeStruct((N, HID), jnp.float32),
            jax.ShapeDtypeStruct((1, WDL), jnp.float32),
        ],
    )(adj, feat, wc0, wc1, wc2, wc3, w1, b1r, w2t, b2r)


def _sc_body(a_hbm, b_hbm, w_hbm, out0, out1,
             a_sh, b_sh, a_v, b_v, p0v, p1v, wsm, tv):
    cid = lax.axis_index("c")
    sid = lax.axis_index("s")
    wid = sid * 2 + cid
    base = wid * PPW

    @pl.when(sid == 0)
    def _stage():
        pltpu.sync_copy(a_hbm, a_sh)
        pltpu.sync_copy(b_hbm, b_sh)

    pltpu.sync_copy(w_hbm, wsm)

    p0s = base

    def find_i(k, acc):
        off_k = (k * (2 * N - 1 - k)) // 2
        return jnp.where(off_k <= p0s, k, acc)

    i0 = lax.fori_loop(1, N, find_i, jnp.int32(0))
    j0 = i0 + 1 + (p0s - (i0 * (2 * N - 1 - i0)) // 2)

    plsc.subcore_barrier()
    pltpu.sync_copy(a_sh, a_v)
    pltpu.sync_copy(b_sh, b_v)
    wregs = [wsm[0, pl.ds(16 * q, 16)] for q in range(NCHK)]
    dlv = wsm[0, pl.ds(HID, 16)]
    sc17 = lax.iota(jnp.int32, 16) * 17
    nm1 = jnp.int32(N - 1)

    ci, cj = i0, j0
    for b in range(NBLK):
        tb = b * (17 * BLK)
        for p in range(BLK):
            i_s = jnp.minimum(ci, nm1)
            j_s = jnp.minimum(cj, nm1)
            acc = jnp.zeros((16,), jnp.float32)
            for q in range(NCHK):
                va = a_v[i_s, pl.ds(16 * q, 16)]
                vb = b_v[j_s, pl.ds(16 * q, 16)]
                acc = acc + jnp.maximum(va + vb, 0.0) * wregs[q]
            plsc.store_scatter(tv, [sc17 + (tb + p)], acc)
            wrap = cj >= nm1
            ci = jnp.where(wrap, ci + 1, ci)
            cj = jnp.where(wrap, ci + 1, cj + 1)
        tot = dlv
        for l in range(16):
            tot = tot + tv[pl.ds(tb + 17 * l, 16)]
        t = jnp.exp(-tot)
        p0 = 1.0 / (1.0 + t)
        p0v[pl.ds(b * BLK, BLK)] = p0
        p1v[pl.ds(b * BLK, BLK)] = 1.0 - p0
    pltpu.sync_copy(p0v, out0.at[pl.ds(base, PPW)])
    pltpu.sync_copy(p1v, out1.at[pl.ds(base, PPW)])


@functools.cache
def _sc_pairs_fn():
    mesh = plsc.VectorSubcoreMesh(core_axis_name="c", subcore_axis_name="s")
    return functools.partial(
        pl.kernel,
        mesh=mesh,
        compiler_params=pltpu.CompilerParams(
            needs_layout_passes=False, skip_device_barrier=True),
        out_type=[
            jax.ShapeDtypeStruct((NPAD,), jnp.float32),
            jax.ShapeDtypeStruct((NPAD,), jnp.float32),
        ],
        scratch_types=[
            pltpu.VMEM_SHARED((N, HID), jnp.float32),
            pltpu.VMEM_SHARED((N, HID), jnp.float32),
            pltpu.VMEM((N, HID), jnp.float32),
            pltpu.VMEM((N, HID), jnp.float32),
            pltpu.VMEM((PPW,), jnp.float32),
            pltpu.VMEM((PPW,), jnp.float32),
            pltpu.VMEM((1, WDL), jnp.float32),
            pltpu.VMEM((NBLK * 16 * 17,), jnp.float32),
        ],
    )(_sc_body)


def kernel(adj_matrix, atom_features, num_atoms, Wc0, Wc1, Wc2, Wc3,
           W1, b1, W2, b2):
    del num_atoms
    adj = adj_matrix[0]
    feat = atom_features[0]
    a_t, b_t, wdl = _tc_tables(
        adj, feat, Wc0, Wc1, Wc2, Wc3,
        W1, b1.reshape(1, HID), W2.T, b2.reshape(1, NCLS))
    out0, out1 = _sc_pairs_fn()(a_t, b_t, wdl)
    probs = jnp.stack([out0[:P], out1[:P]], axis=1)
    idxs = jnp.asarray(_IDXS)
    return (idxs, probs)


# speedup vs baseline: 1.2997x; 1.2997x over previous
import functools

import numpy as np
import jax
import jax.numpy as jnp
from jax import lax
from jax.experimental import pallas as pl
from jax.experimental.pallas import tpu as pltpu
from jax.experimental.pallas import tpu_sc as plsc

N = 100
F = 41
HID = 128
NCLS = 2
P = N * (N - 1) // 2

NW = 32
PPW = 160
NPAD = NW * PPW
BLK = 16
NCHK = HID // 16
NBLK = PPW // BLK
WDL = HID + BLK

_i_np, _j_np = np.triu_indices(N, k=1)
_IDXS = np.stack([_i_np, _j_np], axis=1).astype(np.int32)
_I_PAD = np.zeros((NPAD,), np.int32)
_J_PAD = np.zeros((NPAD,), np.int32)
_I_PAD[:P] = _i_np
_J_PAD[:P] = _j_np


def _tc_body(adj, feat, wc0, wc1, wc2, wc3, w1, b1, w2t, b2,
             a_out, b_out, wdl_out):
    adj_v = adj[...]
    deg = jnp.sum(adj_v, axis=1, keepdims=True)
    na = adj_v / jnp.maximum(deg, 1.0)
    h = feat[...]
    for wc in (wc0, wc1, wc2, wc3):
        h = jnp.maximum(
            jnp.dot(jnp.dot(na, h, preferred_element_type=jnp.float32),
                    wc[...], preferred_element_type=jnp.float32), 0.0)
    w1v = w1[...]
    a_out[...] = jnp.dot(h, w1v[:F], preferred_element_type=jnp.float32) + b1[...]
    b_out[...] = jnp.dot(h, w1v[F:], preferred_element_type=jnp.float32)
    wd_row = w2t[0:1, :] - w2t[1:2, :]
    dltb = b2[0, 0] - b2[0, 1]
    wdl_out[...] = jnp.concatenate(
        [wd_row, jnp.full((1, BLK), dltb, jnp.float32)], axis=1)


def _tc_tables(adj, feat, wc0, wc1, wc2, wc3, w1, b1r, w2t, b2r):
    return pl.pallas_call(
        _tc_body,
        out_shape=[
            jax.ShapeDtypeStruct((N, HID), jnp.float32),
            jax.ShapeDtypeStruct((N, HID), jnp.float32),
            jax.ShapeDtypeStruct((1, WDL), jnp.float32),
        ],
    )(adj, feat, wc0, wc1, wc2, wc3, w1, b1r, w2t, b2r)


def _sc_body(a_hbm, b_hbm, w_hbm, out0, out1,
             a_sh, b_sh, a_v, b_v, p0v, p1v, wsm, tv):
    cid = lax.axis_index("c")
    sid = lax.axis_index("s")
    wid = sid * 2 + cid
    base = wid * PPW

    @pl.when(sid == 0)
    def _stage():
        pltpu.sync_copy(a_hbm, a_sh)
        pltpu.sync_copy(b_hbm, b_sh)

    pltpu.sync_copy(w_hbm, wsm)

    p0s = base

    def find_i(k, acc):
        off_k = (k * (2 * N - 1 - k)) // 2
        return jnp.where(off_k <= p0s, k, acc)

    i0 = lax.fori_loop(1, N, find_i, jnp.int32(0))
    j0 = i0 + 1 + (p0s - (i0 * (2 * N - 1 - i0)) // 2)

    plsc.subcore_barrier()
    pltpu.sync_copy(a_sh, a_v)
    pltpu.sync_copy(b_sh, b_v)
    wregs = [wsm[0, pl.ds(16 * q, 16)] for q in range(NCHK)]
    dlv = wsm[0, pl.ds(HID, 16)]
    sc17 = lax.iota(jnp.int32, 16) * 17
    nm1 = jnp.int32(N - 1)

    def pairfn(p, carry):
        ci, cj = carry
        i_s = jnp.minimum(ci, nm1)
        j_s = jnp.minimum(cj, nm1)
        acc = jnp.zeros((16,), jnp.float32)
        for q in range(NCHK):
            va = a_v[i_s, pl.ds(16 * q, 16)]
            vb = b_v[j_s, pl.ds(16 * q, 16)]
            acc = acc + jnp.maximum(va + vb, 0.0) * wregs[q]
        plsc.store_scatter(tv, [sc17 + p], acc)
        wrap = cj >= nm1
        ci = jnp.where(wrap, ci + 1, ci)
        cj = jnp.where(wrap, ci + 1, cj + 1)
        return ci, cj

    def blk(b, carry):
        carry = lax.fori_loop(0, BLK, pairfn, carry)
        tot = dlv
        for l in range(16):
            tot = tot + tv[pl.ds(17 * l, 16)]
        t = jnp.exp(-tot)
        p0 = 1.0 / (1.0 + t)
        p0v[pl.ds(b * BLK, BLK)] = p0
        p1v[pl.ds(b * BLK, BLK)] = 1.0 - p0
        return carry

    lax.fori_loop(0, NBLK, blk, (i0, j0))
    pltpu.sync_copy(p0v, out0.at[pl.ds(base, PPW)])
    pltpu.sync_copy(p1v, out1.at[pl.ds(base, PPW)])


@functools.cache
def _sc_pairs_fn():
    mesh = plsc.VectorSubcoreMesh(core_axis_name="c", subcore_axis_name="s")
    return functools.partial(
        pl.kernel,
        mesh=mesh,
        compiler_params=pltpu.CompilerParams(
            needs_layout_passes=False, skip_device_barrier=True),
        out_type=[
            jax.ShapeDtypeStruct((NPAD,), jnp.float32),
            jax.ShapeDtypeStruct((NPAD,), jnp.float32),
        ],
        scratch_types=[
            pltpu.VMEM_SHARED((N, HID), jnp.float32),
            pltpu.VMEM_SHARED((N, HID), jnp.float32),
            pltpu.VMEM((N, HID), jnp.float32),
            pltpu.VMEM((N, HID), jnp.float32),
            pltpu.VMEM((PPW,), jnp.float32),
            pltpu.VMEM((PPW,), jnp.float32),
            pltpu.VMEM((1, WDL), jnp.float32),
            pltpu.VMEM((16 * 17,), jnp.float32),
        ],
    )(_sc_body)


def kernel(adj_matrix, atom_features, num_atoms, Wc0, Wc1, Wc2, Wc3,
           W1, b1, W2, b2):
    del num_atoms
    adj = adj_matrix[0]
    feat = atom_features[0]
    a_t, b_t, wdl = _tc_tables(
        adj, feat, Wc0, Wc1, Wc2, Wc3,
        W1, b1.reshape(1, HID), W2.T, b2.reshape(1, NCLS))
    out0, out1 = _sc_pairs_fn()(a_t, b_t, wdl)
    probs = jnp.stack([out0[:P], out1[:P]], axis=1)
    idxs = jnp.asarray(_IDXS)
    return (idxs, probs)


# speedup vs baseline: 1.3454x; 1.0352x over previous
import functools

import numpy as np
import jax
import jax.numpy as jnp
from jax import lax
from jax.experimental import pallas as pl
from jax.experimental.pallas import tpu as pltpu
from jax.experimental.pallas import tpu_sc as plsc

N = 100
F = 41
HID = 128
NCLS = 2
P = N * (N - 1) // 2

NW = 32
PPW = 160
NPAD = NW * PPW
BLK = 16
NCHK = HID // 16
NBLK = PPW // BLK
WDL = HID + BLK

_i_np, _j_np = np.triu_indices(N, k=1)
_IDXS = np.stack([_i_np, _j_np], axis=1).astype(np.int32)
_I_PAD = np.zeros((NPAD,), np.int32)
_J_PAD = np.zeros((NPAD,), np.int32)
_I_PAD[:P] = _i_np
_J_PAD[:P] = _j_np


def _tc_body(adj, feat, wc0, wc1, wc2, wc3, w1, b1, w2t, b2,
             a_out, b_out, wdl_out):
    adj_v = adj[...]
    deg = jnp.sum(adj_v, axis=1, keepdims=True)
    na = adj_v / jnp.maximum(deg, 1.0)
    h = feat[...]
    for wc in (wc0, wc1, wc2, wc3):
        h = jnp.maximum(
            jnp.dot(jnp.dot(na, h, preferred_element_type=jnp.float32),
                    wc[...], preferred_element_type=jnp.float32), 0.0)
    w1v = w1[...]
    a_out[...] = jnp.dot(h, w1v[:F], preferred_element_type=jnp.float32) + b1[...]
    b_out[...] = jnp.dot(h, w1v[F:], preferred_element_type=jnp.float32)
    wd_row = w2t[0:1, :] - w2t[1:2, :]
    dltb = b2[0, 0] - b2[0, 1]
    wdl_out[...] = jnp.concatenate(
        [wd_row, jnp.full((1, BLK), dltb, jnp.float32)], axis=1)


def _tc_tables(adj, feat, wc0, wc1, wc2, wc3, w1, b1r, w2t, b2r):
    return pl.pallas_call(
        _tc_body,
        out_shape=[
            jax.ShapeDtypeStruct((N, HID), jnp.float32),
            jax.ShapeDtypeStruct((N, HID), jnp.float32),
            jax.ShapeDtypeStruct((1, WDL), jnp.float32),
        ],
    )(adj, feat, wc0, wc1, wc2, wc3, w1, b1r, w2t, b2r)


def _sc_body(a_hbm, b_hbm, w_hbm, out0, out1,
             a_sh, b_sh, a_v, b_v, p0v, p1v, wsm, tv):
    cid = lax.axis_index("c")
    sid = lax.axis_index("s")
    wid = sid * 2 + cid
    base = wid * PPW

    @pl.when(sid == 0)
    def _stage_a():
        pltpu.sync_copy(a_hbm, a_sh)

    @pl.when(sid == 1)
    def _stage_b():
        pltpu.sync_copy(b_hbm, b_sh)

    pltpu.sync_copy(w_hbm, wsm)

    p0s = jnp.int32(base)
    iot = lax.iota(jnp.int32, 16)
    i0 = jnp.int32(0)
    for kb in range(7):
        kvec = iot + (16 * kb + 1)
        offv = (kvec * (2 * N - 1 - kvec)) // 2
        i0 = i0 + plsc.all_reduce_population_count(offv <= p0s)[0]
    j0 = i0 + 1 + (p0s - (i0 * (2 * N - 1 - i0)) // 2)

    plsc.subcore_barrier()
    pltpu.sync_copy(a_sh, a_v)
    pltpu.sync_copy(b_sh, b_v)
    wregs = [wsm[0, pl.ds(16 * q, 16)] for q in range(NCHK)]
    dlv = wsm[0, pl.ds(HID, 16)]
    sc17 = lax.iota(jnp.int32, 16) * 17
    nm1 = jnp.int32(N - 1)

    def pairfn(p, carry):
        ci, cj = carry
        i_s = jnp.minimum(ci, nm1)
        j_s = jnp.minimum(cj, nm1)
        acc = jnp.zeros((16,), jnp.float32)
        for q in range(NCHK):
            va = a_v[i_s, pl.ds(16 * q, 16)]
            vb = b_v[j_s, pl.ds(16 * q, 16)]
            acc = acc + jnp.maximum(va + vb, 0.0) * wregs[q]
        plsc.store_scatter(tv, [sc17 + p], acc)
        wrap = cj >= nm1
        ci = jnp.where(wrap, ci + 1, ci)
        cj = jnp.where(wrap, ci + 1, cj + 1)
        return ci, cj

    def blk(b, carry):
        carry = lax.fori_loop(0, BLK, pairfn, carry)
        tot = dlv
        for l in range(16):
            tot = tot + tv[pl.ds(17 * l, 16)]
        t = jnp.exp(-tot)
        p0 = 1.0 / (1.0 + t)
        p0v[pl.ds(b * BLK, BLK)] = p0
        p1v[pl.ds(b * BLK, BLK)] = 1.0 - p0
        return carry

    lax.fori_loop(0, NBLK, blk, (i0, j0))
    pltpu.sync_copy(p0v, out0.at[pl.ds(base, PPW)])
    pltpu.sync_copy(p1v, out1.at[pl.ds(base, PPW)])


@functools.cache
def _sc_pairs_fn():
    mesh = plsc.VectorSubcoreMesh(core_axis_name="c", subcore_axis_name="s")
    return functools.partial(
        pl.kernel,
        mesh=mesh,
        compiler_params=pltpu.CompilerParams(
            needs_layout_passes=False, skip_device_barrier=True),
        out_type=[
            jax.ShapeDtypeStruct((NPAD,), jnp.float32),
            jax.ShapeDtypeStruct((NPAD,), jnp.float32),
        ],
        scratch_types=[
            pltpu.VMEM_SHARED((N, HID), jnp.float32),
            pltpu.VMEM_SHARED((N, HID), jnp.float32),
            pltpu.VMEM((N, HID), jnp.float32),
            pltpu.VMEM((N, HID), jnp.float32),
            pltpu.VMEM((PPW,), jnp.float32),
            pltpu.VMEM((PPW,), jnp.float32),
            pltpu.VMEM((1, WDL), jnp.float32),
            pltpu.VMEM((16 * 17,), jnp.float32),
        ],
    )(_sc_body)


def kernel(adj_matrix, atom_features, num_atoms, Wc0, Wc1, Wc2, Wc3,
           W1, b1, W2, b2):
    del num_atoms
    adj = adj_matrix[0]
    feat = atom_features[0]
    a_t, b_t, wdl = _tc_tables(
        adj, feat, Wc0, Wc1, Wc2, Wc3,
        W1, b1.reshape(1, HID), W2.T, b2.reshape(1, NCLS))
    out0, out1 = _sc_pairs_fn()(a_t, b_t, wdl)
    probs = jnp.stack([out0[:P], out1[:P]], axis=1)
    idxs = jnp.asarray(_IDXS)
    return (idxs, probs)


# speedup vs baseline: 1.3487x; 1.0025x over previous
import functools

import numpy as np
import jax
import jax.numpy as jnp
from jax import lax
from jax.experimental import pallas as pl
from jax.experimental.pallas import tpu as pltpu
from jax.experimental.pallas import tpu_sc as plsc

N = 100
F = 41
HID = 128
NCLS = 2
P = N * (N - 1) // 2

NW = 32
PPW = 160
NPAD = NW * PPW
BLK = 16
NCHK = HID // 16
NBLK = PPW // BLK
WDL = HID + BLK

_i_np, _j_np = np.triu_indices(N, k=1)
_IDXS = np.stack([_i_np, _j_np], axis=1).astype(np.int32)
_I_PAD = np.zeros((NPAD,), np.int32)
_J_PAD = np.zeros((NPAD,), np.int32)
_I_PAD[:P] = _i_np
_J_PAD[:P] = _j_np


def _tc_body(adj, feat, wc0, wc1, wc2, wc3, w1, b1, w2t, b2,
             a_out, b_out, wdl_out):
    adj_v = adj[...]
    deg = jnp.sum(adj_v, axis=1, keepdims=True)
    na = adj_v / jnp.maximum(deg, 1.0)
    h = feat[...]
    for wc in (wc0, wc1, wc2, wc3):
        h = jnp.maximum(
            jnp.dot(jnp.dot(na, h, preferred_element_type=jnp.float32),
                    wc[...], preferred_element_type=jnp.float32), 0.0)
    w1v = w1[...]
    a_out[...] = jnp.dot(h, w1v[:F], preferred_element_type=jnp.float32) + b1[...]
    b_out[...] = jnp.dot(h, w1v[F:], preferred_element_type=jnp.float32)
    wd_row = w2t[0:1, :] - w2t[1:2, :]
    dltb = b2[0, 0] - b2[0, 1]
    wdl_out[...] = jnp.concatenate(
        [wd_row, jnp.full((1, BLK), dltb, jnp.float32)], axis=1)


def _tc_tables(adj, feat, wc0, wc1, wc2, wc3, w1, b1r, w2t, b2r):
    return pl.pallas_call(
        _tc_body,
        out_shape=[
            jax.ShapeDtypeStruct((N, HID), jnp.float32),
            jax.ShapeDtypeStruct((N, HID), jnp.float32),
            jax.ShapeDtypeStruct((1, WDL), jnp.float32),
        ],
    )(adj, feat, wc0, wc1, wc2, wc3, w1, b1r, w2t, b2r)


def _sc_body(a_hbm, b_hbm, w_hbm, out0, out1,
             a_sh, b_sh, a_v, b_v, p0v, p1v, wsm, tv):
    cid = lax.axis_index("c")
    sid = lax.axis_index("s")
    wid = sid * 2 + cid
    base = wid * PPW

    @pl.when(sid == 0)
    def _stage_a():
        pltpu.sync_copy(a_hbm, a_sh)

    @pl.when(sid == 1)
    def _stage_b():
        pltpu.sync_copy(b_hbm, b_sh)

    pltpu.sync_copy(w_hbm, wsm)

    p0s = jnp.int32(base)
    iot = lax.iota(jnp.int32, 16)
    i0 = jnp.int32(0)
    for kb in range(7):
        kvec = iot + (16 * kb + 1)
        offv = (kvec * (2 * N - 1 - kvec)) // 2
        i0 = i0 + plsc.all_reduce_population_count(offv <= p0s)[0]
    j0 = i0 + 1 + (p0s - (i0 * (2 * N - 1 - i0)) // 2)

    plsc.subcore_barrier()
    pltpu.sync_copy(a_sh, a_v)
    pltpu.sync_copy(b_sh, b_v)
    wregs = [wsm[0, pl.ds(16 * q, 16)] for q in range(NCHK)]
    dlv = wsm[0, pl.ds(HID, 16)]
    sc17 = lax.iota(jnp.int32, 16) * 17
    nm1 = jnp.int32(N - 1)

    def pairfn(p, carry):
        ci, cj = carry
        i_s = jnp.minimum(ci, nm1)
        j_s = jnp.minimum(cj, nm1)
        acc = jnp.zeros((16,), jnp.float32)
        for q in range(NCHK):
            va = a_v[i_s, pl.ds(16 * q, 16)]
            vb = b_v[j_s, pl.ds(16 * q, 16)]
            acc = acc + jnp.maximum(va + vb, 0.0) * wregs[q]
        plsc.store_scatter(tv, [sc17 + p], acc)
        wrap = cj >= nm1
        ci = jnp.where(wrap, ci + 1, ci)
        cj = jnp.where(wrap, ci + 1, cj + 1)
        return ci, cj

    def blk(b, carry):
        carry = lax.fori_loop(0, BLK, pairfn, carry)
        tot = dlv
        for l in range(16):
            tot = tot + tv[pl.ds(17 * l, 16)]
        t = jnp.exp(-tot)
        p0 = 1.0 / (1.0 + t)
        p0v[pl.ds(b * BLK, BLK)] = p0
        p1v[pl.ds(b * BLK, BLK)] = 1.0 - p0
        return carry

    lax.fori_loop(0, NBLK, blk, (i0, j0))
    pltpu.sync_copy(p0v, out0.at[pl.ds(base, PPW)])
    pltpu.sync_copy(p1v, out1.at[pl.ds(base, PPW)])


@functools.cache
def _sc_pairs_fn():
    mesh = plsc.VectorSubcoreMesh(core_axis_name="c", subcore_axis_name="s")
    return functools.partial(
        pl.kernel,
        mesh=mesh,
        compiler_params=pltpu.CompilerParams(
            needs_layout_passes=False, skip_device_barrier=True),
        out_type=[
            jax.ShapeDtypeStruct((NPAD,), jnp.float32),
            jax.ShapeDtypeStruct((NPAD,), jnp.float32),
        ],
        scratch_types=[
            pltpu.VMEM_SHARED((N, HID), jnp.float32),
            pltpu.VMEM_SHARED((N, HID), jnp.float32),
            pltpu.VMEM((N, HID), jnp.float32),
            pltpu.VMEM((N, HID), jnp.float32),
            pltpu.VMEM((PPW,), jnp.float32),
            pltpu.VMEM((PPW,), jnp.float32),
            pltpu.VMEM((1, WDL), jnp.float32),
            pltpu.VMEM((16 * 17,), jnp.float32),
        ],
    )(_sc_body)


def kernel(adj_matrix, atom_features, num_atoms, Wc0, Wc1, Wc2, Wc3,
           W1, b1, W2, b2):
    zero = jnp.asarray(num_atoms - num_atoms, dtype=jnp.int32)
    adj = adj_matrix[0]
    feat = atom_features[0]
    a_t, b_t, wdl = _tc_tables(
        adj, feat, Wc0, Wc1, Wc2, Wc3,
        W1, b1.reshape(1, HID), W2.T, b2.reshape(1, NCLS))
    out0, out1 = _sc_pairs_fn()(a_t, b_t, wdl)
    probs = jnp.stack([out0[:P], out1[:P]], axis=1)
    idxs = jnp.asarray(_IDXS) + zero
    return (idxs, probs)
